# baseline (speedup 1.0000x reference)
import functools

import jax
import jax.numpy as jnp
from jax import lax
from jax.experimental import pallas as pl
from jax.experimental.pallas import tpu as pltpu
from jax.experimental.pallas import tpu_sc as plsc

N = 10000
E = 320000
F = 128
LAT = 64

NC = 2
NS = 16
NW = NC * NS
CHUNK = 128
CPW0 = 98
CPW1 = 60
E_PAD = NS * (CPW0 + CPW1) * CHUNK
M = 10240
RPS = M // NS

_SC_MESH = plsc.VectorSubcoreMesh(core_axis_name="c", subcore_axis_name="s")


def _sc_degree(dstc, zeros1):
    @functools.partial(
        pl.kernel,
        mesh=_SC_MESH,
        out_type=jax.ShapeDtypeStruct((NC, M), jnp.float32),
        scratch_types=[
            pltpu.VMEM((CHUNK,), jnp.int32),
            pltpu.VMEM((CHUNK,), jnp.float32),
            pltpu.VMEM_SHARED((M,), jnp.float32),
        ],
    )
    def k(dst_hbm, zeros_hbm, out_hbm, didx, ones_v, dacc):
        c = lax.axis_index("c")
        s = lax.axis_index("s")
        r0 = s * RPS
        cpw_d = (CPW0 + CPW1) // 2
        base = (s * NC + c) * cpw_d

        @pl.loop(0, CHUNK, step=16)
        def _(i):
            ones_v[pl.ds(i, 16)] = jnp.ones((16,), jnp.float32)

        pltpu.sync_copy(zeros_hbm.at[pl.ds(r0, RPS)], dacc.at[pl.ds(r0, RPS)])
        plsc.subcore_barrier()

        @pl.loop(0, cpw_d)
        def _(kk):
            pltpu.sync_copy(dst_hbm.at[base + kk], didx)
            pltpu.sync_copy(ones_v, dacc.at[didx], add=True)

        plsc.subcore_barrier()
        pltpu.sync_copy(dacc.at[pl.ds(r0, RPS)], out_hbm.at[c].at[pl.ds(r0, RPS)])

    return k(dstc, zeros1)


def _sc_propagate(table, srcc, dstc, zeros2):
    @functools.partial(
        pl.kernel,
        mesh=_SC_MESH,
        out_type=jax.ShapeDtypeStruct((NC, M, F), jnp.float32),
        scratch_types=[
            pltpu.VMEM((CHUNK,), jnp.int32),
            pltpu.VMEM((CHUNK,), jnp.int32),
            pltpu.VMEM((CHUNK,), jnp.int32),
            pltpu.VMEM((CHUNK,), jnp.int32),
            pltpu.VMEM((CHUNK, F), jnp.float32),
            pltpu.VMEM((CHUNK, F), jnp.float32),
            pltpu.VMEM_SHARED((M, F), jnp.float32),
            pltpu.SemaphoreType.DMA,
            pltpu.SemaphoreType.DMA,
            pltpu.SemaphoreType.DMA,
        ],
    )
    def k(table_hbm, src_hbm, dst_hbm, zeros_hbm, out_hbm,
          sidxA, didxA, sidxB, didxB, rowsA, rowsB, acc, semG, semSA, semSB):
        c = lax.axis_index("c")
        s = lax.axis_index("s")
        r0 = s * RPS
        cpw_c = jnp.where(c == 0, CPW0, CPW1)
        base = c * (NS * CPW0) + s * cpw_c

        pltpu.sync_copy(zeros_hbm.at[pl.ds(r0, RPS)], acc.at[pl.ds(r0, RPS)])
        plsc.subcore_barrier()

        def step(kk, sidx, didx, rows, semS):
            @pl.when(kk >= 2)
            def _():
                pltpu.make_async_copy(rows, acc.at[didx], semS).wait()

            pltpu.sync_copy(src_hbm.at[base + kk], sidx)
            pltpu.sync_copy(dst_hbm.at[base + kk], didx)
            pltpu.async_copy(table_hbm.at[sidx], rows, semG).wait()
            pltpu.async_copy(rows, acc.at[didx], semS, add=True)

        @pl.loop(0, CPW0 + CPW1, step=2)
        def _(kk):
            @pl.when(kk < cpw_c)
            def _():
                step(kk, sidxA, didxA, rowsA, semSA)
                step(kk + 1, sidxB, didxB, rowsB, semSB)

        pltpu.make_async_copy(rowsA, acc.at[didxA], semSA).wait()
        pltpu.make_async_copy(rowsB, acc.at[didxB], semSB).wait()
        plsc.subcore_barrier()
        pltpu.sync_copy(acc.at[pl.ds(r0, RPS)], out_hbm.at[c].at[pl.ds(r0, RPS)])

    return k(table, srcc, dstc, zeros2)


BLK = 2000
_GRID = N // BLK


def _k1_matmul(x, W1):
    def body(x_ref, w_ref, o_ref):
        o_ref[...] = jnp.dot(x_ref[...], w_ref[...],
                             preferred_element_type=jnp.float32)

    return pl.pallas_call(
        body,
        grid=(_GRID,),
        in_specs=[
            pl.BlockSpec((BLK, F), lambda i: (i, 0)),
            pl.BlockSpec((F, F), lambda i: (0, 0)),
        ],
        out_specs=pl.BlockSpec((BLK, F), lambda i: (i, 0)),
        out_shape=jax.ShapeDtypeStruct((N, F), jnp.float32),
    )(x, W1)


def _k2_dinv_scale(h1, degp):
    deg3 = degp.reshape(NC, M, 1)

    def body(h_ref, d0_ref, d1_ref, hp_ref, dinv_ref):
        deg = d0_ref[0] + d1_ref[0] + 1.0
        dinv = lax.rsqrt(deg)
        dinv_ref[...] = dinv
        hp_ref[...] = h_ref[...] * dinv

    return pl.pallas_call(
        body,
        grid=(_GRID,),
        in_specs=[
            pl.BlockSpec((BLK, F), lambda i: (i, 0)),
            pl.BlockSpec((1, BLK, 1), lambda i: (0, i, 0)),
            pl.BlockSpec((1, BLK, 1), lambda i: (1, i, 0)),
        ],
        out_specs=[
            pl.BlockSpec((BLK, F), lambda i: (i, 0)),
            pl.BlockSpec((BLK, 1), lambda i: (i, 0)),
        ],
        out_shape=[
            jax.ShapeDtypeStruct((N, F), jnp.float32),
            jax.ShapeDtypeStruct((N, 1), jnp.float32),
        ],
    )(h1, deg3, deg3)


def _k3_layer1_finish(acc1, h1p, dinv, b1, Wc):
    def body(a0_ref, a1_ref, hp_ref, dinv_ref, b_ref, w_ref, o_ref):
        pre = (a0_ref[0] + a1_ref[0] + hp_ref[...]) * dinv_ref[...] + b_ref[...]
        h = jnp.maximum(pre, 0.0)
        o_ref[...] = jnp.dot(h, w_ref[...],
                             preferred_element_type=jnp.float32) * dinv_ref[...]

    return pl.pallas_call(
        body,
        grid=(_GRID,),
        in_specs=[
            pl.BlockSpec((1, BLK, F), lambda i: (0, i, 0)),
            pl.BlockSpec((1, BLK, F), lambda i: (1, i, 0)),
            pl.BlockSpec((BLK, F), lambda i: (i, 0)),
            pl.BlockSpec((BLK, 1), lambda i: (i, 0)),
            pl.BlockSpec((F,), lambda i: (0,)),
            pl.BlockSpec((F, F), lambda i: (0, 0)),
        ],
        out_specs=pl.BlockSpec((BLK, F), lambda i: (i, 0)),
        out_shape=jax.ShapeDtypeStruct((N, F), jnp.float32),
    )(acc1, acc1, h1p, dinv, b1, Wc)


def _k4_layer2_finish(acc2, hc, dinv, bc):
    def body(a0_ref, a1_ref, hc_ref, dinv_ref, b_ref, o_ref):
        o_ref[...] = ((a0_ref[0] + a1_ref[0] + hc_ref[...]) * dinv_ref[...]
                      + b_ref[...])

    return pl.pallas_call(
        body,
        grid=(_GRID,),
        in_specs=[
            pl.BlockSpec((1, BLK, F), lambda i: (0, i, 0)),
            pl.BlockSpec((1, BLK, F), lambda i: (1, i, 0)),
            pl.BlockSpec((BLK, F), lambda i: (i, 0)),
            pl.BlockSpec((BLK, 1), lambda i: (i, 0)),
            pl.BlockSpec((F,), lambda i: (0,)),
        ],
        out_specs=pl.BlockSpec((BLK, F), lambda i: (i, 0)),
        out_shape=jax.ShapeDtypeStruct((N, F), jnp.float32),
    )(acc2, acc2, hc, dinv, bc)


def kernel(x, edge_index, W1, b1, Wmu, bmu, Wlv, blv):
    src = edge_index[0]
    dst = edge_index[1]
    pad = E_PAD - E
    srcc = jnp.concatenate(
        [src, jnp.zeros((pad,), jnp.int32)]).reshape(-1, CHUNK)
    dstc = jnp.concatenate(
        [dst, jnp.full((pad,), N, jnp.int32)]).reshape(-1, CHUNK)
    zeros1 = jnp.zeros((M,), jnp.float32)
    zeros2 = jnp.zeros((M, F), jnp.float32)
    Wc = jnp.concatenate([Wmu, Wlv], axis=1)
    bc = jnp.concatenate([bmu, blv])

    degp = _sc_degree(dstc, zeros1)
    h1 = _k1_matmul(x, W1)
    h1p, dinv = _k2_dinv_scale(h1, degp)
    acc1 = _sc_propagate(h1p, srcc, dstc, zeros2)
    hc = _k3_layer1_finish(acc1, h1p, dinv, b1, Wc)
    acc2 = _sc_propagate(hc, srcc, dstc, zeros2)
    out2 = _k4_layer2_finish(acc2, hc, dinv, bc)
    return (out2[:, :LAT], out2[:, LAT:])


# speedup vs baseline: 1.0336x; 1.0336x over previous
import functools

import jax
import jax.numpy as jnp
from jax import lax
from jax.experimental import pallas as pl
from jax.experimental.pallas import tpu as pltpu
from jax.experimental.pallas import tpu_sc as plsc

N = 10000
E = 320000
F = 128
LAT = 64

NC = 2
NS = 16
NW = NC * NS
CHUNK = 128
CPW0 = 104
CPW1 = 54
E_PAD = NS * (CPW0 + CPW1) * CHUNK
M = 10240
RPS = M // NS

_SC_MESH = plsc.VectorSubcoreMesh(core_axis_name="c", subcore_axis_name="s")


def _sc_degree(dstc, zeros1):
    @functools.partial(
        pl.kernel,
        mesh=_SC_MESH,
        out_type=jax.ShapeDtypeStruct((NC, M), jnp.float32),
        scratch_types=[
            pltpu.VMEM((CHUNK,), jnp.int32),
            pltpu.VMEM((CHUNK,), jnp.float32),
            pltpu.VMEM_SHARED((M,), jnp.float32),
        ],
    )
    def k(dst_hbm, zeros_hbm, out_hbm, didx, ones_v, dacc):
        c = lax.axis_index("c")
        s = lax.axis_index("s")
        r0 = s * RPS
        cpw_d = (CPW0 + CPW1) // 2
        base = (s * NC + c) * cpw_d

        @pl.loop(0, CHUNK, step=16)
        def _(i):
            ones_v[pl.ds(i, 16)] = jnp.ones((16,), jnp.float32)

        pltpu.sync_copy(zeros_hbm.at[pl.ds(r0, RPS)], dacc.at[pl.ds(r0, RPS)])
        plsc.subcore_barrier()

        @pl.loop(0, cpw_d)
        def _(kk):
            pltpu.sync_copy(dst_hbm.at[base + kk], didx)
            pltpu.sync_copy(ones_v, dacc.at[didx], add=True)

        plsc.subcore_barrier()
        pltpu.sync_copy(dacc.at[pl.ds(r0, RPS)], out_hbm.at[c].at[pl.ds(r0, RPS)])

    return k(dstc, zeros1)


def _sc_propagate(table, srcc, dstc, zeros2):
    @functools.partial(
        pl.kernel,
        mesh=_SC_MESH,
        out_type=jax.ShapeDtypeStruct((NC, M, F), jnp.float32),
        scratch_types=[
            pltpu.VMEM((CHUNK,), jnp.int32),
            pltpu.VMEM((CHUNK,), jnp.int32),
            pltpu.VMEM((CHUNK,), jnp.int32),
            pltpu.VMEM((CHUNK,), jnp.int32),
            pltpu.VMEM((CHUNK, F), jnp.float32),
            pltpu.VMEM((CHUNK, F), jnp.float32),
            pltpu.VMEM_SHARED((M, F), jnp.float32),
            pltpu.SemaphoreType.DMA,
            pltpu.SemaphoreType.DMA,
            pltpu.SemaphoreType.DMA,
        ],
    )
    def k(table_hbm, src_hbm, dst_hbm, zeros_hbm, out_hbm,
          sidxA, didxA, sidxB, didxB, rowsA, rowsB, acc, semG, semSA, semSB):
        c = lax.axis_index("c")
        s = lax.axis_index("s")
        r0 = s * RPS
        cpw_c = jnp.where(c == 0, CPW0, CPW1)
        base = c * (NS * CPW0) + s * cpw_c

        pltpu.sync_copy(zeros_hbm.at[pl.ds(r0, RPS)], acc.at[pl.ds(r0, RPS)])
        plsc.subcore_barrier()

        def step(kk, sidx, didx, rows, semS):
            @pl.when(kk >= 2)
            def _():
                pltpu.make_async_copy(rows, acc.at[didx], semS).wait()

            pltpu.sync_copy(src_hbm.at[base + kk], sidx)
            pltpu.sync_copy(dst_hbm.at[base + kk], didx)
            pltpu.async_copy(table_hbm.at[sidx], rows, semG).wait()
            pltpu.async_copy(rows, acc.at[didx], semS, add=True)

        @pl.loop(0, CPW0 + CPW1, step=2)
        def _(kk):
            @pl.when(kk < cpw_c)
            def _():
                step(kk, sidxA, didxA, rowsA, semSA)
                step(kk + 1, sidxB, didxB, rowsB, semSB)

        pltpu.make_async_copy(rowsA, acc.at[didxA], semSA).wait()
        pltpu.make_async_copy(rowsB, acc.at[didxB], semSB).wait()
        plsc.subcore_barrier()
        pltpu.sync_copy(acc.at[pl.ds(r0, RPS)], out_hbm.at[c].at[pl.ds(r0, RPS)])

    return k(table, srcc, dstc, zeros2)


BLK = 2000
_GRID = N // BLK


def _k1_matmul(x, W1):
    def body(x_ref, w_ref, o_ref):
        o_ref[...] = jnp.dot(x_ref[...], w_ref[...],
                             preferred_element_type=jnp.float32)

    return pl.pallas_call(
        body,
        grid=(_GRID,),
        in_specs=[
            pl.BlockSpec((BLK, F), lambda i: (i, 0)),
            pl.BlockSpec((F, F), lambda i: (0, 0)),
        ],
        out_specs=pl.BlockSpec((BLK, F), lambda i: (i, 0)),
        out_shape=jax.ShapeDtypeStruct((N, F), jnp.float32),
    )(x, W1)


def _k2_dinv_scale(h1, degp):
    deg3 = degp.reshape(NC, M, 1)

    def body(h_ref, d0_ref, d1_ref, hp_ref, dinv_ref):
        deg = d0_ref[0] + d1_ref[0] + 1.0
        dinv = lax.rsqrt(deg)
        dinv_ref[...] = dinv
        hp_ref[...] = h_ref[...] * dinv

    return pl.pallas_call(
        body,
        grid=(_GRID,),
        in_specs=[
            pl.BlockSpec((BLK, F), lambda i: (i, 0)),
            pl.BlockSpec((1, BLK, 1), lambda i: (0, i, 0)),
            pl.BlockSpec((1, BLK, 1), lambda i: (1, i, 0)),
        ],
        out_specs=[
            pl.BlockSpec((BLK, F), lambda i: (i, 0)),
            pl.BlockSpec((BLK, 1), lambda i: (i, 0)),
        ],
        out_shape=[
            jax.ShapeDtypeStruct((N, F), jnp.float32),
            jax.ShapeDtypeStruct((N, 1), jnp.float32),
        ],
    )(h1, deg3, deg3)


def _k3_layer1_finish(acc1, h1p, dinv, b1, Wc):
    def body(a0_ref, a1_ref, hp_ref, dinv_ref, b_ref, w_ref, o_ref):
        pre = (a0_ref[0] + a1_ref[0] + hp_ref[...]) * dinv_ref[...] + b_ref[...]
        h = jnp.maximum(pre, 0.0)
        o_ref[...] = jnp.dot(h, w_ref[...],
                             preferred_element_type=jnp.float32) * dinv_ref[...]

    return pl.pallas_call(
        body,
        grid=(_GRID,),
        in_specs=[
            pl.BlockSpec((1, BLK, F), lambda i: (0, i, 0)),
            pl.BlockSpec((1, BLK, F), lambda i: (1, i, 0)),
            pl.BlockSpec((BLK, F), lambda i: (i, 0)),
            pl.BlockSpec((BLK, 1), lambda i: (i, 0)),
            pl.BlockSpec((F,), lambda i: (0,)),
            pl.BlockSpec((F, F), lambda i: (0, 0)),
        ],
        out_specs=pl.BlockSpec((BLK, F), lambda i: (i, 0)),
        out_shape=jax.ShapeDtypeStruct((N, F), jnp.float32),
    )(acc1, acc1, h1p, dinv, b1, Wc)


def _k4_layer2_finish(acc2, hc, dinv, bc):
    def body(a0_ref, a1_ref, hc_ref, dinv_ref, b_ref, o_ref):
        o_ref[...] = ((a0_ref[0] + a1_ref[0] + hc_ref[...]) * dinv_ref[...]
                      + b_ref[...])

    return pl.pallas_call(
        body,
        grid=(_GRID,),
        in_specs=[
            pl.BlockSpec((1, BLK, F), lambda i: (0, i, 0)),
            pl.BlockSpec((1, BLK, F), lambda i: (1, i, 0)),
            pl.BlockSpec((BLK, F), lambda i: (i, 0)),
            pl.BlockSpec((BLK, 1), lambda i: (i, 0)),
            pl.BlockSpec((F,), lambda i: (0,)),
        ],
        out_specs=pl.BlockSpec((BLK, F), lambda i: (i, 0)),
        out_shape=jax.ShapeDtypeStruct((N, F), jnp.float32),
    )(acc2, acc2, hc, dinv, bc)


def kernel(x, edge_index, W1, b1, Wmu, bmu, Wlv, blv):
    src = edge_index[0]
    dst = edge_index[1]
    pad = E_PAD - E
    srcc = jnp.concatenate(
        [src, jnp.zeros((pad,), jnp.int32)]).reshape(-1, CHUNK)
    dstc = jnp.concatenate(
        [dst, jnp.full((pad,), N, jnp.int32)]).reshape(-1, CHUNK)
    zeros1 = jnp.zeros((M,), jnp.float32)
    zeros2 = jnp.zeros((M, F), jnp.float32)
    Wc = jnp.concatenate([Wmu, Wlv], axis=1)
    bc = jnp.concatenate([bmu, blv])

    degp = _sc_degree(dstc, zeros1)
    h1 = _k1_matmul(x, W1)
    h1p, dinv = _k2_dinv_scale(h1, degp)
    acc1 = _sc_propagate(h1p, srcc, dstc, zeros2)
    hc = _k3_layer1_finish(acc1, h1p, dinv, b1, Wc)
    acc2 = _sc_propagate(hc, srcc, dstc, zeros2)
    out2 = _k4_layer2_finish(acc2, hc, dinv, bc)
    return (out2[:, :LAT], out2[:, LAT:])
